# baseline (speedup 1.0000x reference)
import functools

import jax
import jax.numpy as jnp
from jax import lax
from jax.experimental import pallas as pl
from jax.experimental.pallas import tpu as pltpu
from jax.experimental.pallas import tpu_sc as plsc

_NC = 2
_NS = 16
_LANES = 16
_CHUNK = 4000


@functools.lru_cache(maxsize=None)
def _make_sc_segsum(n_edges: int, n_nodes: int, chunk: int):
    n_workers = _NC * _NS
    e_per_w = n_edges // n_workers
    n_chunks = e_per_w // chunk
    assert e_per_w * n_workers == n_edges
    assert n_chunks * chunk == e_per_w and n_chunks % 2 == 0
    assert chunk % _LANES == 0 and chunk % 8 == 0 and e_per_w % 8 == 0

    seg = -(-n_nodes // (_NS * 8)) * 8
    n_pad = seg * _NS

    mesh = plsc.VectorSubcoreMesh(core_axis_name="c", subcore_axis_name="s")

    @functools.partial(
        pl.kernel,
        mesh=mesh,
        out_type=(
            jax.ShapeDtypeStruct((n_workers * n_pad,), jnp.float32),
            jax.ShapeDtypeStruct((_NC * n_pad,), jnp.float32),
        ),
        scratch_types=[
            pltpu.VMEM((chunk + _LANES,), jnp.int32),
            pltpu.VMEM((chunk + _LANES,), jnp.int32),
            pltpu.VMEM((chunk,), jnp.float32),
            pltpu.VMEM((chunk,), jnp.float32),
            pltpu.VMEM((seg,), jnp.float32),
            pltpu.VMEM((seg,), jnp.float32),
            pltpu.VMEM_SHARED((_NS * _LANES,), jnp.int32),
            pltpu.SemaphoreType.DMA,
            pltpu.SemaphoreType.DMA,
            pltpu.VMEM((n_pad,), jnp.float32),
        ],
        compiler_params=pltpu.CompilerParams(needs_layout_passes=False),
    )
    def segsum(edge_src_hbm, vals_hbm, stage_hbm, out_hbm,
               ib0, ib1, vb0, vb1, mrg, tmp, rng_sh, sem0, sem1, acc):
        cid = lax.axis_index("c")
        sid = lax.axis_index("s")
        wid = cid * _NS + sid

        def chunk_copies(j, ib, vb, sem):
            base = wid * e_per_w + j * chunk
            return (
                pltpu.make_async_copy(edge_src_hbm.at[pl.ds(base, chunk)],
                                      ib.at[pl.ds(0, chunk)], sem),
                pltpu.make_async_copy(vals_hbm.at[pl.ds(base, chunk)],
                                      vb, sem),
            )

        def start(j, ib, vb, sem):
            a, b = chunk_copies(j, ib, vb, sem)
            a.start()
            b.start()

        def wait(j, ib, vb, sem):
            a, b = chunk_copies(j, ib, vb, sem)
            a.wait()
            b.wait()

        start(0, ib0, vb0, sem0)

        zero16 = jnp.zeros((_LANES,), jnp.float32)

        @plsc.parallel_loop(0, n_pad, _LANES, unroll=8)
        def _(o):
            acc[pl.ds(o, _LANES)] = zero16

        lane = lax.iota(jnp.int32, _LANES)
        m15 = lane == (_LANES - 1)

        def process(ib, vb):
            @plsc.parallel_loop(0, chunk, _LANES, unroll=8)
            def _(o):
                idx = ib[pl.ds(o, _LANES)]
                nxt = ib[pl.ds(o + 1, _LANES)]
                val = vb[pl.ds(o, _LANES)]
                s = plsc.cumsum(val)
                mb = idx != nxt
                plsc.addupdate_scatter(acc, [idx], s, mask=mb | m15)
                plsc.addupdate_scatter(acc, [nxt], -s, mask=mb & ~m15)

        def pair_body(j2, carry):
            j0 = 2 * j2
            start(j0 + 1, ib1, vb1, sem1)
            wait(j0, ib0, vb0, sem0)
            process(ib0, vb0)

            @pl.when(j0 + 2 < n_chunks)
            def _():
                start(j0 + 2, ib0, vb0, sem0)

            wait(j0 + 1, ib1, vb1, sem1)
            process(ib1, vb1)
            return carry

        lax.fori_loop(0, n_chunks // 2, pair_body, 0)

        pltpu.sync_copy(edge_src_hbm.at[pl.ds(wid * e_per_w, _LANES)],
                        ib0.at[pl.ds(0, _LANES)])
        idx_first = ib0[pl.ds(0, _LANES)][0]
        idx_last = ib1[pl.ds(chunk - _LANES, _LANES)][_LANES - 1]

        def page_of(x):
            p = jnp.int32(0)
            for t in range(1, _NS):
                p = p + jnp.where(x >= t * seg, 1, 0).astype(jnp.int32)
            return p

        p_lo = page_of(idx_first)
        p_hi = page_of(idx_last)

        def flush_body(p, carry):
            pltpu.sync_copy(
                acc.at[pl.ds(p * seg, seg)],
                stage_hbm.at[pl.ds(wid * n_pad + p * seg, seg)])
            return carry

        lax.fori_loop(p_lo, p_hi + 1, flush_body, 0)

        ib1[pl.ds(0, _LANES)] = jnp.where(
            lane == 0, p_lo, jnp.where(lane == 1, p_hi, 0)).astype(jnp.int32)
        pltpu.sync_copy(ib1.at[pl.ds(0, _LANES)],
                        rng_sh.at[pl.ds(sid * _LANES, _LANES)])
        plsc.subcore_barrier()
        pltpu.sync_copy(rng_sh, ib0.at[pl.ds(0, _NS * _LANES)])

        @plsc.parallel_loop(0, seg, _LANES, unroll=8)
        def _(o):
            mrg[pl.ds(o, _LANES)] = zero16

        col = sid * seg
        for t in range(_NS):
            rng_t = ib0[pl.ds(t * _LANES, _LANES)]
            lo_t = rng_t[0]
            hi_t = rng_t[1]

            @pl.when((lo_t <= sid) & (sid <= hi_t))
            def _():
                pltpu.sync_copy(
                    stage_hbm.at[pl.ds((cid * _NS + t) * n_pad + col, seg)],
                    tmp)

                @plsc.parallel_loop(0, seg, _LANES, unroll=8)
                def _(o):
                    mrg[pl.ds(o, _LANES)] = (mrg[pl.ds(o, _LANES)]
                                             + tmp[pl.ds(o, _LANES)])

        pltpu.sync_copy(mrg, out_hbm.at[pl.ds(cid * n_pad + col, seg)])

    return segsum, n_pad


def kernel(edge_src, switch, species):
    n_edges = edge_src.shape[0]
    n_nodes = species.shape[0]
    seg, n_pad = _make_sc_segsum(n_edges, n_nodes, _CHUNK)
    _, partials = seg(edge_src, switch)
    partials = partials.reshape(_NC, n_pad)
    return (partials[0] + partials[1])[:n_nodes]
